# Initial kernel scaffold; baseline (speedup 1.0000x reference)
import jax
import jax.numpy as jnp
from jax.experimental import pallas as pl


def kernel(x, pi, edge_index, batch, Wl0, bl0, Wr0, Wl1, bl1, Wr1, Wl2, bl2, Wr2, fc1_W, fc1_b, ln1_g, ln1_b, fcp_W, fcp_b, lnp_g, lnp_b, fc2_W, fc2_b, ln2_g, ln2_b):
    raise NotImplementedError("write your pallas kernel here")


# speedup vs baseline: 4.6393x; 4.6393x over previous
import functools

import jax
import jax.numpy as jnp
from jax import lax
from jax.experimental import pallas as pl
from jax.experimental.pallas import tpu as pltpu
from jax.experimental.pallas import tpu_sc as plsc

N = 10000
E = 320000
G = 64
DF = 128
DN = 256
DG = 512
DP = 16
DPE = 64
DT = 64

NP = 10240
NCORE = 2
NSUB = 16
NW = NCORE * NSUB
K = 128
EPT = E // NSUB
NCH = -(-EPT // K)
EPAD = NCH * K
RPT = NP // NSUB


def _zero_fill(rows_v, W):
  def zrow(i, _):
    for v in range(W // 16):
      rows_v[i, pl.ds(v * 16, 16)] = jnp.zeros((16,), jnp.float32)
    return 0
  lax.fori_loop(0, K, zrow, 0)


@functools.lru_cache(maxsize=None)
def _make_agg():
  W = DN // 2
  mesh = plsc.VectorSubcoreMesh(core_axis_name="c", subcore_axis_name="s")
  out_type = jax.ShapeDtypeStruct((NCORE, NP, W), jnp.float32)
  scratch = [
      pltpu.VMEM((K,), jnp.int32),
      pltpu.VMEM((K,), jnp.int32),
      pltpu.VMEM((K, W), jnp.float32),
      pltpu.VMEM_SHARED((NP, W), jnp.float32),
      pltpu.SemaphoreType.DMA,
  ]

  @functools.partial(pl.kernel, mesh=mesh, out_type=out_type,
                     scratch_types=scratch)
  def agg_kernel(hs, srcs, dsts, out, src_v, dst_v, rows_v, acc_sh, sem):
    c = lax.axis_index("c")
    s = lax.axis_index("s")

    _zero_fill(rows_v, W)
    for j in range(RPT // K):
      pltpu.sync_copy(rows_v, acc_sh.at[pl.ds(s * RPT + j * K, K)])

    plsc.subcore_barrier()

    def chunk(j, _):
      pltpu.sync_copy(srcs.at[c, s, j], src_v)
      pltpu.sync_copy(dsts.at[s, j], dst_v)
      pltpu.async_copy(hs.at[src_v], rows_v, sem).wait()
      pltpu.sync_copy(rows_v, acc_sh.at[dst_v], add=True)
      return 0
    lax.fori_loop(0, NCH, chunk, 0)

    plsc.subcore_barrier()

    pltpu.sync_copy(acc_sh.at[pl.ds(s * RPT, RPT)],
                    out.at[c, pl.ds(s * RPT, RPT)])

  return agg_kernel


EPT0 = E // NW
NCH0 = -(-EPT0 // K)
EPAD0 = NCH0 * K


@functools.lru_cache(maxsize=None)
def _make_agg0():
  W = DF
  mesh = plsc.VectorSubcoreMesh(core_axis_name="c", subcore_axis_name="s")
  out_type = [jax.ShapeDtypeStruct((NCORE, NP, W), jnp.float32),
              jax.ShapeDtypeStruct((NCORE, NP), jnp.float32)]
  scratch = [
      pltpu.VMEM((NCH0, K), jnp.int32),
      pltpu.VMEM((NCH0, K), jnp.int32),
      pltpu.VMEM((K, W), jnp.float32),
      pltpu.VMEM((K,), jnp.float32),
      pltpu.VMEM_SHARED((NP, W), jnp.float32),
      pltpu.VMEM_SHARED((NP,), jnp.float32),
      pltpu.SemaphoreType.DMA,
  ]

  @functools.partial(pl.kernel, mesh=mesh, out_type=out_type,
                     scratch_types=scratch)
  def agg0_kernel(hs, srcs, dsts, out, deg_out, src_v, dst_v, rows_v,
                  ones_v, acc_sh, deg_sh, sem):
    c = lax.axis_index("c")
    s = lax.axis_index("s")
    w = c * NSUB + s

    pltpu.sync_copy(srcs.at[w], src_v)
    pltpu.sync_copy(dsts.at[w], dst_v)

    _zero_fill(rows_v, W)
    for v in range(K // 16):
      ones_v[pl.ds(v * 16, 16)] = jnp.zeros((16,), jnp.float32)
    for j in range(RPT // K):
      pltpu.sync_copy(rows_v, acc_sh.at[pl.ds(s * RPT + j * K, K)])
    for j in range(RPT // K):
      pltpu.sync_copy(ones_v, deg_sh.at[pl.ds(s * RPT + j * K, K)])
    for v in range(K // 16):
      ones_v[pl.ds(v * 16, 16)] = jnp.ones((16,), jnp.float32)

    plsc.subcore_barrier()

    def chunk(j, _):
      pltpu.async_copy(hs.at[src_v.at[j]], rows_v, sem).wait()
      pltpu.sync_copy(rows_v, acc_sh.at[dst_v.at[j]], add=True)
      pltpu.sync_copy(ones_v, deg_sh.at[dst_v.at[j]], add=True)
      return 0
    lax.fori_loop(0, NCH0, chunk, 0)

    plsc.subcore_barrier()

    pltpu.sync_copy(acc_sh.at[pl.ds(s * RPT, RPT)],
                    out.at[c, pl.ds(s * RPT, RPT)])
    pltpu.sync_copy(deg_sh.at[pl.ds(s * RPT, RPT)],
                    deg_out.at[c, pl.ds(s * RPT, RPT)])

  return agg0_kernel


_RB = 400


def _layer0_body(a_ref, x_ref, degp_ref, wl_ref, wr_ref, bl_ref, out_ref,
                 deg_ref):
  deg = degp_ref[0] + degp_ref[1]
  deg_ref[...] = deg
  rinv = 1.0 / jnp.maximum(deg, 1.0)
  agg = jnp.dot(a_ref[0] + a_ref[1], wl_ref[...],
                preferred_element_type=jnp.float32)
  hr = jnp.dot(x_ref[...], wr_ref[...], preferred_element_type=jnp.float32)
  h = jnp.maximum(agg * rinv + bl_ref[...] + hr, 0.0)
  out_ref[0] = h[:, :128]
  out_ref[1] = h[:, 128:]


_layer0 = pl.pallas_call(
    _layer0_body,
    grid=(N // _RB,),
    in_specs=[
        pl.BlockSpec((NCORE, _RB, DF), lambda i: (0, i, 0)),
        pl.BlockSpec((_RB, DF), lambda i: (i, 0)),
        pl.BlockSpec((NCORE, _RB, 1), lambda i: (0, i, 0)),
        pl.BlockSpec((DF, DN), lambda i: (0, 0)),
        pl.BlockSpec((DF, DN), lambda i: (0, 0)),
        pl.BlockSpec((1, DN), lambda i: (0, 0)),
    ],
    out_specs=[
        pl.BlockSpec((NCORE, _RB, 128), lambda i: (0, i, 0)),
        pl.BlockSpec((_RB, 1), lambda i: (i, 0)),
    ],
    out_shape=[
        jax.ShapeDtypeStruct((NCORE, NP, 128), jnp.float32),
        jax.ShapeDtypeStruct((NP, 1), jnp.float32),
    ],
)


def _layer12_body(a_ref, h_ref, deg_ref, wl_ref, wr_ref, bl_ref, out_ref):
  W = DN // 2
  rinv = 1.0 / jnp.maximum(deg_ref[...], 1.0)
  agg = (jnp.dot(a_ref[0], wl_ref[:W, :], preferred_element_type=jnp.float32)
         + jnp.dot(a_ref[1], wl_ref[W:, :],
                   preferred_element_type=jnp.float32))
  hr = (jnp.dot(h_ref[0], wr_ref[:W, :], preferred_element_type=jnp.float32)
        + jnp.dot(h_ref[1], wr_ref[W:, :],
                  preferred_element_type=jnp.float32))
  h = jnp.maximum(agg * rinv + bl_ref[...] + hr, 0.0)
  out_ref[0] = h[:, :128]
  out_ref[1] = h[:, 128:]


_layer12 = pl.pallas_call(
    _layer12_body,
    grid=(N // _RB,),
    in_specs=[
        pl.BlockSpec((NCORE, _RB, DN // 2), lambda i: (0, i, 0)),
        pl.BlockSpec((NCORE, _RB, DN // 2), lambda i: (0, i, 0)),
        pl.BlockSpec((_RB, 1), lambda i: (i, 0)),
        pl.BlockSpec((DN, DN), lambda i: (0, 0)),
        pl.BlockSpec((DN, DN), lambda i: (0, 0)),
        pl.BlockSpec((1, DN), lambda i: (0, 0)),
    ],
    out_specs=pl.BlockSpec((NCORE, _RB, 128), lambda i: (0, i, 0)),
    out_shape=jax.ShapeDtypeStruct((NCORE, NP, 128), jnp.float32),
)


RPW = NP // NW
PCH = 32
NSEG = 6


@functools.lru_cache(maxsize=None)
def _make_pool():
  mesh = plsc.VectorSubcoreMesh(core_axis_name="c", subcore_axis_name="s")
  out_type = jax.ShapeDtypeStruct((NW, G, NSEG * 128), jnp.float32)
  scratch = (
      [pltpu.VMEM((G + 1, NSEG * 128), jnp.float32),
       pltpu.VMEM((RPW + 16,), jnp.int32)]
      + [pltpu.VMEM((PCH, 128), jnp.float32) for _ in range(NSEG)]
  )

  @functools.partial(pl.kernel, mesh=mesh, out_type=out_type,
                     scratch_types=scratch)
  def pool_kernel(h1, h2, h3, batch_hbm, out, tbl_v, batch_v, *chunks):
    c = lax.axis_index("c")
    s = lax.axis_index("s")
    w = s * NCORE + c
    base = w * RPW
    pltpu.sync_copy(batch_hbm.at[pl.ds(base, RPW)], batch_v.at[pl.ds(0, RPW)])

    neg = jnp.full((16,), -jnp.inf, jnp.float32)

    def trow(i, _):
      for v in range(NSEG * 128 // 16):
        tbl_v[i, pl.ds(v * 16, 16)] = neg
      return 0
    lax.fori_loop(0, G + 1, trow, 0)

    def pchunk(ci, _):
      r0 = base + ci * PCH
      for k, (href, half) in enumerate(
          ((h1, 0), (h1, 1), (h2, 0), (h2, 1), (h3, 0), (h3, 1))):
        pltpu.sync_copy(href.at[half, pl.ds(r0, PCH)], chunks[k])

      def prow(r, _):
        g = batch_v[pl.ds(ci * PCH + r, 16)][0]
        for k in range(NSEG):
          for v in range(8):
            o = v * 16
            cur = tbl_v[g, pl.ds(k * 128 + o, 16)]
            tbl_v[g, pl.ds(k * 128 + o, 16)] = jnp.maximum(
                cur, chunks[k][r, pl.ds(o, 16)])
        return 0
      lax.fori_loop(0, PCH, prow, 0)
      return 0
    lax.fori_loop(0, RPW // PCH, pchunk, 0)

    pltpu.sync_copy(tbl_v.at[pl.ds(0, G)], out.at[w])

  return pool_kernel


def _ln(x, g, b, eps=1e-5):
  mu = jnp.mean(x, axis=-1, keepdims=True)
  var = jnp.mean((x - mu) ** 2, axis=-1, keepdims=True)
  return (x - mu) / jnp.sqrt(var + eps) * g + b


def _readout_body(tbl_ref, pi_ref, fc1w_ref, fc1b_ref, ln1g_ref, ln1b_ref,
                  fcpw_ref, fcpb_ref, lnpg_ref, lnpb_ref, fc2wg_ref,
                  fc2wp_ref, fc2b_ref, ln2g_ref, ln2b_ref, out_ref):
  pooled = jnp.max(tbl_ref[...], axis=0)
  g = jnp.dot(pooled, fc1w_ref[...],
              preferred_element_type=jnp.float32) + fc1b_ref[...]
  g = jnp.maximum(_ln(g, ln1g_ref[...], ln1b_ref[...]), 0.0)
  p = jnp.dot(pi_ref[...], fcpw_ref[...],
              preferred_element_type=jnp.float32) + fcpb_ref[...]
  p = jnp.maximum(_ln(p, lnpg_ref[...], lnpb_ref[...]), 0.0)
  o = (jnp.dot(g, fc2wg_ref[...], preferred_element_type=jnp.float32)
       + jnp.dot(p, fc2wp_ref[...], preferred_element_type=jnp.float32)
       + fc2b_ref[...])
  out_ref[...] = _ln(o, ln2g_ref[...], ln2b_ref[...])


_readout = pl.pallas_call(
    _readout_body,
    out_shape=jax.ShapeDtypeStruct((G, DT), jnp.float32),
)


def kernel(x, pi, edge_index, batch, Wl0, bl0, Wr0, Wl1, bl1, Wr1, Wl2, bl2,
           Wr2, fc1_W, fc1_b, ln1_g, ln1_b, fcp_W, fcp_b, lnp_g, lnp_b,
           fc2_W, fc2_b, ln2_g, ln2_b):
  src = edge_index[0].astype(jnp.int32)
  dst = edge_index[1].astype(jnp.int32)

  npad = EPAD - EPT
  pad_src = (jnp.arange(npad, dtype=jnp.int32) * 131 % N)[None, :]
  pad_dst = N + (jnp.arange(npad, dtype=jnp.int32) % (NP - N))[None, :]
  src_t = jnp.concatenate(
      [src.reshape(NSUB, EPT), jnp.broadcast_to(pad_src, (NSUB, npad))], 1)
  dst_t = jnp.concatenate(
      [dst.reshape(NSUB, EPT), jnp.broadcast_to(pad_dst, (NSUB, npad))], 1)
  src_t = src_t.reshape(NSUB, NCH, K)
  dst_idx = dst_t.reshape(NSUB, NCH, K)
  src_idx = jnp.stack([src_t, src_t + NP])

  npad0 = EPAD0 - EPT0
  pad_src0 = (jnp.arange(npad0, dtype=jnp.int32) * 131 % N)[None, :]
  pad_dst0 = N + (jnp.arange(npad0, dtype=jnp.int32) % (NP - N))[None, :]
  src_idx0 = jnp.concatenate(
      [src.reshape(NW, EPT0), jnp.broadcast_to(pad_src0, (NW, npad0))],
      1).reshape(NW, NCH0, K)
  dst_idx0 = jnp.concatenate(
      [dst.reshape(NW, EPT0), jnp.broadcast_to(pad_dst0, (NW, npad0))],
      1).reshape(NW, NCH0, K)

  batch_p = jnp.concatenate(
      [batch.astype(jnp.int32), jnp.full((NP - N,), G, jnp.int32)])

  agg0, degp = _make_agg0()(x, src_idx0, dst_idx0)
  h1, deg2 = _layer0(agg0, x, degp.reshape(NCORE, NP, 1), Wl0, Wr0,
                     bl0.reshape(1, DN))

  agg = _make_agg()
  agg1 = agg(h1.reshape(NCORE * NP, DN // 2), src_idx, dst_idx)
  h2 = _layer12(agg1, h1, deg2, Wl1, Wr1, bl1.reshape(1, DN))

  agg2 = agg(h2.reshape(NCORE * NP, DN // 2), src_idx, dst_idx)
  h3 = _layer12(agg2, h2, deg2, Wl2, Wr2, bl2.reshape(1, DN))

  tables = _make_pool()(h1, h2, h3, batch_p)

  return _readout(
      tables, pi, fc1_W, fc1_b.reshape(1, DG), ln1_g.reshape(1, DG),
      ln1_b.reshape(1, DG), fcp_W, fcp_b.reshape(1, DPE),
      lnp_g.reshape(1, DPE), lnp_b.reshape(1, DPE), fc2_W[:DG],
      fc2_W[DG:], fc2_b.reshape(1, DT), ln2_g.reshape(1, DT),
      ln2_b.reshape(1, DT))


# speedup vs baseline: 6.4544x; 1.3912x over previous
import functools

import jax
import jax.numpy as jnp
from jax import lax
from jax.experimental import pallas as pl
from jax.experimental.pallas import tpu as pltpu
from jax.experimental.pallas import tpu_sc as plsc

N = 10000
E = 320000
G = 64
DF = 128
DN = 256
DG = 512
DP = 16
DPE = 64
DT = 64

NP = 10240
NCORE = 2
NSUB = 16
NW = NCORE * NSUB
K = 128
SLAB = 16
EPT = E // NSUB
NCH = SLAB * (-(-EPT // (SLAB * K)))
EPAD = NCH * K
RPT = NP // NSUB


def _zero_fill(rows_v, W):
  def zrow(i, _):
    for v in range(W // 16):
      rows_v[i, pl.ds(v * 16, 16)] = jnp.zeros((16,), jnp.float32)
    return 0
  lax.fori_loop(0, K, zrow, 0)


@functools.lru_cache(maxsize=None)
def _make_agg():
  W = DN // 2
  mesh = plsc.VectorSubcoreMesh(core_axis_name="c", subcore_axis_name="s")
  out_type = jax.ShapeDtypeStruct((NCORE, NP, W), jnp.float32)
  scratch = [
      pltpu.VMEM((SLAB, K), jnp.int32),
      pltpu.VMEM((SLAB, K), jnp.int32),
      pltpu.VMEM((2, K, W), jnp.float32),
      pltpu.VMEM_SHARED((NP, W), jnp.float32),
      pltpu.SemaphoreType.DMA,
      pltpu.SemaphoreType.DMA,
      pltpu.SemaphoreType.DMA,
      pltpu.SemaphoreType.DMA,
  ]

  @functools.partial(pl.kernel, mesh=mesh, out_type=out_type,
                     scratch_types=scratch)
  def agg_kernel(hs, srcs, dsts, out, src_v, dst_v, rows_v, acc_sh,
                 gsem0, gsem1, ssem0, ssem1):
    c = lax.axis_index("c")
    s = lax.axis_index("s")

    _zero_fill(rows_v.at[0], W)
    for j in range(RPT // K):
      pltpu.sync_copy(rows_v.at[0], acc_sh.at[pl.ds(s * RPT + j * K, K)])

    plsc.subcore_barrier()

    def slab(t, _):
      pltpu.sync_copy(srcs.at[c, s, pl.ds(t * SLAB, SLAB)], src_v)
      pltpu.sync_copy(dsts.at[s, pl.ds(t * SLAB, SLAB)], dst_v)

      def pair(q, _):
        a, b = 2 * q, 2 * q + 1
        g0 = pltpu.async_copy(hs.at[src_v.at[a]], rows_v.at[0], gsem0)
        g1 = pltpu.async_copy(hs.at[src_v.at[b]], rows_v.at[1], gsem1)
        g0.wait()
        s0 = pltpu.async_copy(rows_v.at[0], acc_sh.at[dst_v.at[a]], ssem0,
                              add=True)
        g1.wait()
        s1 = pltpu.async_copy(rows_v.at[1], acc_sh.at[dst_v.at[b]], ssem1,
                              add=True)
        s0.wait()
        s1.wait()
        return 0
      lax.fori_loop(0, SLAB // 2, pair, 0)
      return 0
    lax.fori_loop(0, NCH // SLAB, slab, 0)

    plsc.subcore_barrier()

    pltpu.sync_copy(acc_sh.at[pl.ds(s * RPT, RPT)],
                    out.at[c, pl.ds(s * RPT, RPT)])

  return agg_kernel


EPT0 = E // NW
NCH0 = 2 * (-(-EPT0 // (2 * K)))
EPAD0 = NCH0 * K


@functools.lru_cache(maxsize=None)
def _make_agg0():
  W = DF
  mesh = plsc.VectorSubcoreMesh(core_axis_name="c", subcore_axis_name="s")
  out_type = [jax.ShapeDtypeStruct((NCORE, NP, W), jnp.float32),
              jax.ShapeDtypeStruct((NCORE, NP), jnp.float32)]
  scratch = [
      pltpu.VMEM((SLAB, K), jnp.int32),
      pltpu.VMEM((SLAB, K), jnp.int32),
      pltpu.VMEM((2, K, W), jnp.float32),
      pltpu.VMEM((K,), jnp.float32),
      pltpu.VMEM_SHARED((NP, W), jnp.float32),
      pltpu.VMEM_SHARED((NP,), jnp.float32),
      pltpu.SemaphoreType.DMA,
      pltpu.SemaphoreType.DMA,
      pltpu.SemaphoreType.DMA,
      pltpu.SemaphoreType.DMA,
      pltpu.SemaphoreType.DMA,
      pltpu.SemaphoreType.DMA,
  ]

  @functools.partial(pl.kernel, mesh=mesh, out_type=out_type,
                     scratch_types=scratch)
  def agg0_kernel(hs, srcs, dsts, out, deg_out, src_v, dst_v, rows_v,
                  ones_v, acc_sh, deg_sh, gsem0, gsem1, ssem0, ssem1,
                  dsem0, dsem1):
    c = lax.axis_index("c")
    s = lax.axis_index("s")
    w = c * NSUB + s

    _zero_fill(rows_v.at[0], W)
    for v in range(K // 16):
      ones_v[pl.ds(v * 16, 16)] = jnp.zeros((16,), jnp.float32)
    for j in range(RPT // K):
      pltpu.sync_copy(rows_v.at[0], acc_sh.at[pl.ds(s * RPT + j * K, K)])
    for j in range(RPT // K):
      pltpu.sync_copy(ones_v, deg_sh.at[pl.ds(s * RPT + j * K, K)])
    for v in range(K // 16):
      ones_v[pl.ds(v * 16, 16)] = jnp.ones((16,), jnp.float32)

    plsc.subcore_barrier()

    def slab(t, _):
      pltpu.sync_copy(srcs.at[w, pl.ds(t * SLAB, SLAB)], src_v)
      pltpu.sync_copy(dsts.at[w, pl.ds(t * SLAB, SLAB)], dst_v)

      def pair(q, _):
        a, b = 2 * q, 2 * q + 1
        g0 = pltpu.async_copy(hs.at[src_v.at[a]], rows_v.at[0], gsem0)
        g1 = pltpu.async_copy(hs.at[src_v.at[b]], rows_v.at[1], gsem1)
        g0.wait()
        s0 = pltpu.async_copy(rows_v.at[0], acc_sh.at[dst_v.at[a]], ssem0,
                              add=True)
        d0 = pltpu.async_copy(ones_v, deg_sh.at[dst_v.at[a]], dsem0,
                              add=True)
        g1.wait()
        s1 = pltpu.async_copy(rows_v.at[1], acc_sh.at[dst_v.at[b]], ssem1,
                              add=True)
        d1 = pltpu.async_copy(ones_v, deg_sh.at[dst_v.at[b]], dsem1,
                              add=True)
        s0.wait()
        d0.wait()
        s1.wait()
        d1.wait()
        return 0
      lax.fori_loop(0, SLAB // 2, pair, 0)
      return 0
    lax.fori_loop(0, NCH0 // SLAB, slab, 0)

    plsc.subcore_barrier()

    pltpu.sync_copy(acc_sh.at[pl.ds(s * RPT, RPT)],
                    out.at[c, pl.ds(s * RPT, RPT)])
    pltpu.sync_copy(deg_sh.at[pl.ds(s * RPT, RPT)],
                    deg_out.at[c, pl.ds(s * RPT, RPT)])

  return agg0_kernel


_RB = 400


def _layer0_body(a_ref, x_ref, degp_ref, wl_ref, wr_ref, bl_ref, out_ref,
                 deg_ref):
  deg = degp_ref[0] + degp_ref[1]
  deg_ref[...] = deg
  rinv = 1.0 / jnp.maximum(deg, 1.0)
  agg = jnp.dot(a_ref[0] + a_ref[1], wl_ref[...],
                preferred_element_type=jnp.float32)
  hr = jnp.dot(x_ref[...], wr_ref[...], preferred_element_type=jnp.float32)
  h = jnp.maximum(agg * rinv + bl_ref[...] + hr, 0.0)
  out_ref[0] = h[:, :128]
  out_ref[1] = h[:, 128:]


_layer0 = pl.pallas_call(
    _layer0_body,
    grid=(N // _RB,),
    in_specs=[
        pl.BlockSpec((NCORE, _RB, DF), lambda i: (0, i, 0)),
        pl.BlockSpec((_RB, DF), lambda i: (i, 0)),
        pl.BlockSpec((NCORE, _RB, 1), lambda i: (0, i, 0)),
        pl.BlockSpec((DF, DN), lambda i: (0, 0)),
        pl.BlockSpec((DF, DN), lambda i: (0, 0)),
        pl.BlockSpec((1, DN), lambda i: (0, 0)),
    ],
    out_specs=[
        pl.BlockSpec((NCORE, _RB, 128), lambda i: (0, i, 0)),
        pl.BlockSpec((_RB, 1), lambda i: (i, 0)),
    ],
    out_shape=[
        jax.ShapeDtypeStruct((NCORE, NP, 128), jnp.float32),
        jax.ShapeDtypeStruct((NP, 1), jnp.float32),
    ],
)


def _layer12_body(a_ref, h_ref, deg_ref, wl_ref, wr_ref, bl_ref, out_ref):
  W = DN // 2
  rinv = 1.0 / jnp.maximum(deg_ref[...], 1.0)
  agg = (jnp.dot(a_ref[0], wl_ref[:W, :], preferred_element_type=jnp.float32)
         + jnp.dot(a_ref[1], wl_ref[W:, :],
                   preferred_element_type=jnp.float32))
  hr = (jnp.dot(h_ref[0], wr_ref[:W, :], preferred_element_type=jnp.float32)
        + jnp.dot(h_ref[1], wr_ref[W:, :],
                  preferred_element_type=jnp.float32))
  h = jnp.maximum(agg * rinv + bl_ref[...] + hr, 0.0)
  out_ref[0] = h[:, :128]
  out_ref[1] = h[:, 128:]


_layer12 = pl.pallas_call(
    _layer12_body,
    grid=(N // _RB,),
    in_specs=[
        pl.BlockSpec((NCORE, _RB, DN // 2), lambda i: (0, i, 0)),
        pl.BlockSpec((NCORE, _RB, DN // 2), lambda i: (0, i, 0)),
        pl.BlockSpec((_RB, 1), lambda i: (i, 0)),
        pl.BlockSpec((DN, DN), lambda i: (0, 0)),
        pl.BlockSpec((DN, DN), lambda i: (0, 0)),
        pl.BlockSpec((1, DN), lambda i: (0, 0)),
    ],
    out_specs=pl.BlockSpec((NCORE, _RB, 128), lambda i: (0, i, 0)),
    out_shape=jax.ShapeDtypeStruct((NCORE, NP, 128), jnp.float32),
)


RPW = NP // NW
PCH = 32
NSEG = 6


@functools.lru_cache(maxsize=None)
def _make_pool():
  mesh = plsc.VectorSubcoreMesh(core_axis_name="c", subcore_axis_name="s")
  out_type = jax.ShapeDtypeStruct((NW, G, NSEG * 128), jnp.float32)
  scratch = (
      [pltpu.VMEM((G + 1, NSEG * 128), jnp.float32),
       pltpu.VMEM((RPW + 16,), jnp.int32)]
      + [pltpu.VMEM((PCH, 128), jnp.float32) for _ in range(NSEG)]
  )

  @functools.partial(pl.kernel, mesh=mesh, out_type=out_type,
                     scratch_types=scratch)
  def pool_kernel(h1, h2, h3, batch_hbm, out, tbl_v, batch_v, *chunks):
    c = lax.axis_index("c")
    s = lax.axis_index("s")
    w = s * NCORE + c
    base = w * RPW
    pltpu.sync_copy(batch_hbm.at[pl.ds(base, RPW)], batch_v.at[pl.ds(0, RPW)])

    neg = jnp.full((16,), -jnp.inf, jnp.float32)

    def trow(i, _):
      for v in range(NSEG * 128 // 16):
        tbl_v[i, pl.ds(v * 16, 16)] = neg
      return 0
    lax.fori_loop(0, G + 1, trow, 0)

    def pchunk(ci, _):
      r0 = base + ci * PCH
      for k, (href, half) in enumerate(
          ((h1, 0), (h1, 1), (h2, 0), (h2, 1), (h3, 0), (h3, 1))):
        pltpu.sync_copy(href.at[half, pl.ds(r0, PCH)], chunks[k])

      def prow(r, _):
        g = batch_v[pl.ds(ci * PCH + r, 16)][0]
        for k in range(NSEG):
          for v in range(8):
            o = v * 16
            cur = tbl_v[g, pl.ds(k * 128 + o, 16)]
            tbl_v[g, pl.ds(k * 128 + o, 16)] = jnp.maximum(
                cur, chunks[k][r, pl.ds(o, 16)])
        return 0
      lax.fori_loop(0, PCH, prow, 0)
      return 0
    lax.fori_loop(0, RPW // PCH, pchunk, 0)

    pltpu.sync_copy(tbl_v.at[pl.ds(0, G)], out.at[w])

  return pool_kernel


def _ln(x, g, b, eps=1e-5):
  mu = jnp.mean(x, axis=-1, keepdims=True)
  var = jnp.mean((x - mu) ** 2, axis=-1, keepdims=True)
  return (x - mu) / jnp.sqrt(var + eps) * g + b


def _readout_body(tbl_ref, pi_ref, fc1w_ref, fc1b_ref, ln1g_ref, ln1b_ref,
                  fcpw_ref, fcpb_ref, lnpg_ref, lnpb_ref, fc2wg_ref,
                  fc2wp_ref, fc2b_ref, ln2g_ref, ln2b_ref, out_ref):
  pooled = jnp.max(tbl_ref[...], axis=0)
  g = jnp.dot(pooled, fc1w_ref[...],
              preferred_element_type=jnp.float32) + fc1b_ref[...]
  g = jnp.maximum(_ln(g, ln1g_ref[...], ln1b_ref[...]), 0.0)
  p = jnp.dot(pi_ref[...], fcpw_ref[...],
              preferred_element_type=jnp.float32) + fcpb_ref[...]
  p = jnp.maximum(_ln(p, lnpg_ref[...], lnpb_ref[...]), 0.0)
  o = (jnp.dot(g, fc2wg_ref[...], preferred_element_type=jnp.float32)
       + jnp.dot(p, fc2wp_ref[...], preferred_element_type=jnp.float32)
       + fc2b_ref[...])
  out_ref[...] = _ln(o, ln2g_ref[...], ln2b_ref[...])


_readout = pl.pallas_call(
    _readout_body,
    out_shape=jax.ShapeDtypeStruct((G, DT), jnp.float32),
)


def kernel(x, pi, edge_index, batch, Wl0, bl0, Wr0, Wl1, bl1, Wr1, Wl2, bl2,
           Wr2, fc1_W, fc1_b, ln1_g, ln1_b, fcp_W, fcp_b, lnp_g, lnp_b,
           fc2_W, fc2_b, ln2_g, ln2_b):
  src = edge_index[0].astype(jnp.int32)
  dst = edge_index[1].astype(jnp.int32)

  npad = EPAD - EPT
  pad_src = (jnp.arange(npad, dtype=jnp.int32) * 131 % N)[None, :]
  pad_dst = N + (jnp.arange(npad, dtype=jnp.int32) % (NP - N))[None, :]
  src_t = jnp.concatenate(
      [src.reshape(NSUB, EPT), jnp.broadcast_to(pad_src, (NSUB, npad))], 1)
  dst_t = jnp.concatenate(
      [dst.reshape(NSUB, EPT), jnp.broadcast_to(pad_dst, (NSUB, npad))], 1)
  src_t = src_t.reshape(NSUB, NCH, K)
  dst_idx = dst_t.reshape(NSUB, NCH, K)
  src_idx = jnp.stack([src_t, src_t + NP])

  npad0 = EPAD0 - EPT0
  pad_src0 = (jnp.arange(npad0, dtype=jnp.int32) * 131 % N)[None, :]
  pad_dst0 = N + (jnp.arange(npad0, dtype=jnp.int32) % (NP - N))[None, :]
  src_idx0 = jnp.concatenate(
      [src.reshape(NW, EPT0), jnp.broadcast_to(pad_src0, (NW, npad0))],
      1).reshape(NW, NCH0, K)
  dst_idx0 = jnp.concatenate(
      [dst.reshape(NW, EPT0), jnp.broadcast_to(pad_dst0, (NW, npad0))],
      1).reshape(NW, NCH0, K)

  batch_p = jnp.concatenate(
      [batch.astype(jnp.int32), jnp.full((NP - N,), G, jnp.int32)])

  agg0, degp = _make_agg0()(x, src_idx0, dst_idx0)
  h1, deg2 = _layer0(agg0, x, degp.reshape(NCORE, NP, 1), Wl0, Wr0,
                     bl0.reshape(1, DN))

  agg = _make_agg()
  agg1 = agg(h1.reshape(NCORE * NP, DN // 2), src_idx, dst_idx)
  h2 = _layer12(agg1, h1, deg2, Wl1, Wr1, bl1.reshape(1, DN))

  agg2 = agg(h2.reshape(NCORE * NP, DN // 2), src_idx, dst_idx)
  h3 = _layer12(agg2, h2, deg2, Wl2, Wr2, bl2.reshape(1, DN))

  tables = _make_pool()(h1, h2, h3, batch_p)

  return _readout(
      tables, pi, fc1_W, fc1_b.reshape(1, DG), ln1_g.reshape(1, DG),
      ln1_b.reshape(1, DG), fcp_W, fcp_b.reshape(1, DPE),
      lnp_g.reshape(1, DPE), lnp_b.reshape(1, DPE), fc2_W[:DG],
      fc2_W[DG:], fc2_b.reshape(1, DT), ln2_g.reshape(1, DT),
      ln2_b.reshape(1, DT))
